# baseline (speedup 1.0000x reference)
import functools

import jax
import jax.numpy as jnp
from jax import lax
from jax.experimental import pallas as pl
from jax.experimental.pallas import tpu as pltpu
from jax.experimental.pallas import tpu_sc as plsc

_G = 512
_NB = 1000
_K = 120
_NEG = -1e30
_HI = jax.lax.Precision.HIGHEST


def _ln(x, g, b, eps=1e-5):
    m = jnp.mean(x, axis=-1, keepdims=True)
    v = jnp.mean((x - m) ** 2, axis=-1, keepdims=True)
    return (x - m) / jnp.sqrt(v + eps) * g + b


def _gelu(x):
    return 0.5 * x * (1.0 + lax.erf(x * (2.0 ** -0.5)))


def _pre0_body(x_ref, atom_ref, vn_ref, bond_ref, hin_ref, gmsg_ref):
    xi = x_ref[:, 0]
    oh = (xi[:, None] == lax.broadcasted_iota(jnp.int32, (1, 128), 1)
          ).astype(jnp.float32)
    h0 = jnp.dot(oh, atom_ref[...], preferred_element_type=jnp.float32,
                 precision=_HI)
    hin = h0 + vn_ref[0:1, :]
    hin_ref[...] = hin
    for a in range(5):
        gmsg_ref[a] = _gelu(hin + bond_ref[a:a + 1, :])


def _pre0(x2, atom_pad, vn0, bond_pad, N, H):
    nblk = N // _NB
    return pl.pallas_call(
        _pre0_body,
        grid=(nblk,),
        in_specs=[
            pl.BlockSpec((_NB, 1), lambda i: (i, 0)),
            pl.BlockSpec((128, H), lambda i: (0, 0)),
            pl.BlockSpec((1, H), lambda i: (0, 0)),
            pl.BlockSpec((8, H), lambda i: (0, 0)),
        ],
        out_specs=[
            pl.BlockSpec((_NB, H), lambda i: (i, 0)),
            pl.BlockSpec((5, _NB, H), lambda i: (0, i, 0)),
        ],
        out_shape=[
            jax.ShapeDtypeStruct((N, H), jnp.float32),
            jax.ShapeDtypeStruct((5, N, H), jnp.float32),
        ],
    )(x2, atom_pad, vn0, bond_pad)


def _pre_body(b_ref, hprev_ref, vn_ref, bond_ref, hin_ref, gmsg_ref):
    bi = b_ref[:, 0]
    oh = (bi[:, None] == lax.broadcasted_iota(jnp.int32, (1, _G), 1)
          ).astype(jnp.float32)
    hvn = jnp.dot(oh, vn_ref[...], preferred_element_type=jnp.float32,
                  precision=_HI)
    hin = hprev_ref[...] + hvn
    hin_ref[...] = hin
    for a in range(5):
        gmsg_ref[a] = _gelu(hin + bond_ref[a:a + 1, :])


def _pre(b2, hprev, vn, bond_pad, N, H):
    nblk = N // _NB
    return pl.pallas_call(
        _pre_body,
        grid=(nblk,),
        in_specs=[
            pl.BlockSpec((_NB, 1), lambda i: (i, 0)),
            pl.BlockSpec((_NB, H), lambda i: (i, 0)),
            pl.BlockSpec((_G, H), lambda i: (0, 0)),
            pl.BlockSpec((8, H), lambda i: (0, 0)),
        ],
        out_specs=[
            pl.BlockSpec((_NB, H), lambda i: (i, 0)),
            pl.BlockSpec((5, _NB, H), lambda i: (0, i, 0)),
        ],
        out_shape=[
            jax.ShapeDtypeStruct((N, H), jnp.float32),
            jax.ShapeDtypeStruct((5, N, H), jnp.float32),
        ],
    )(b2, hprev, vn, bond_pad)


def _npad(N):
    return 128 * ((N + 1 + 127) // 128)


def _edge_chunks(E):
    return 8 * (-(-E // (32 * _K * 8)))


def _edge_agg(tab, gidx, dst, zsrc, N, H, E):
    mesh = plsc.VectorSubcoreMesh(core_axis_name="c", subcore_axis_name="s",
                                  num_cores=2, num_subcores=16)
    npad = _npad(N)
    rpt = npad // 16
    ch = _edge_chunks(E)

    @functools.partial(
        pl.kernel,
        out_type=jax.ShapeDtypeStruct((2, npad, H), jnp.float32),
        mesh=mesh,
        scratch_types=[
            pltpu.VMEM_SHARED((npad, H), jnp.float32),
            [pltpu.VMEM((_K,), jnp.int32) for _ in range(3)],
            [pltpu.VMEM((_K,), jnp.int32) for _ in range(3)],
            [pltpu.VMEM((_K, H), jnp.float32) for _ in range(3)],
            [pltpu.SemaphoreType.DMA for _ in range(3)],
            [pltpu.SemaphoreType.DMA for _ in range(3)],
            [pltpu.SemaphoreType.DMA for _ in range(3)],
        ],
    )
    def k(tab_hbm, gidx_hbm, dst_hbm, zsrc_hbm, out_hbm,
          acc, gv, dv, rows, si, sd, sg):
        c = lax.axis_index("c")
        s = lax.axis_index("s")
        w = c * 16 + s
        base = w * ch * _K
        pltpu.sync_copy(zsrc_hbm, acc.at[pl.ds(s * rpt, rpt)])
        plsc.subcore_barrier()

        def load_idx(j):
            u = j % 3
            a = pltpu.async_copy(gidx_hbm.at[pl.ds(base + j * _K, _K)],
                                 gv[u], si[u])
            b = pltpu.async_copy(dst_hbm.at[pl.ds(base + j * _K, _K)],
                                 dv[u], sd[u])
            return a, b

        def start_gather(j, ia):
            u = j % 3
            ia.wait()
            return pltpu.async_copy(tab_hbm.at[gv[u]], rows[u], sg[u])

        pend_i = {0: load_idx(0), 1: load_idx(1)}
        pend_g = {0: start_gather(0, pend_i[0][0])}
        if ch > 1:
            pend_g[1] = start_gather(1, pend_i[1][0])
        if ch > 2:
            pend_i[2] = load_idx(2)
        for j in range(ch):
            u = j % 3
            pend_g[j].wait()
            pend_i[j][1].wait()
            if j + 2 < ch:
                pend_g[j + 2] = start_gather(j + 2, pend_i[j + 2][0])
            pltpu.sync_copy(rows[u], acc.at[dv[u]], add=True)
            if j + 3 < ch:
                pend_i[j + 3] = load_idx(j + 3)

        plsc.subcore_barrier()
        pltpu.sync_copy(acc.at[pl.ds(s * rpt, rpt)],
                        out_hbm.at[c, pl.ds(s * rpt, rpt)])

    return k(tab, gidx, dst, zsrc)


def _post_body(hin_ref, part_ref, w1_ref, b1_ref, g1_ref, be1_ref,
               w2_ref, b2_ref, gn_ref, bn_ref, er_ref, out_ref, *, final):
    hin = hin_ref[...]
    agg = part_ref[0].astype(jnp.float32) + part_ref[1].astype(jnp.float32)
    t = er_ref[0:1, :] * hin + agg
    z = jnp.dot(t, w1_ref[...], preferred_element_type=jnp.float32)
    z = z + b1_ref[0:1, :]
    z = _ln(z, g1_ref[0:1, :], be1_ref[0:1, :])
    z = _gelu(z)
    z = jnp.dot(z, w2_ref[...], preferred_element_type=jnp.float32)
    z = z + b2_ref[0:1, :]
    z = _ln(z, gn_ref[0:1, :], bn_ref[0:1, :])
    if not final:
        z = _gelu(z)
    out_ref[...] = z + hin


def _post(hin, partial, conv, norm, final, N, H):
    nblk = N // _NB
    er = jnp.ones((1, H), jnp.float32) * (1.0 + conv['eps'])
    v = lambda a: a.reshape(1, -1).astype(jnp.float32)
    full = lambda shape: pl.BlockSpec(shape, lambda i: (0, 0))
    return pl.pallas_call(
        functools.partial(_post_body, final=final),
        grid=(nblk,),
        in_specs=[
            pl.BlockSpec((_NB, H), lambda i: (i, 0)),
            pl.BlockSpec((2, _NB, H), lambda i: (0, i, 0)),
            full((H, 4 * H)),
            full((1, 4 * H)),
            full((1, 4 * H)),
            full((1, 4 * H)),
            full((4 * H, H)),
            full((1, H)),
            full((1, H)),
            full((1, H)),
            full((1, H)),
        ],
        out_specs=pl.BlockSpec((_NB, H), lambda i: (i, 0)),
        out_shape=jax.ShapeDtypeStruct((N, H), jnp.float32),
    )(hin, partial, conv['W1'], v(conv['b1']), v(conv['g1']), v(conv['be1']),
      conv['W2'], v(conv['b2']), v(norm['g']), v(norm['b']), er)


def _segmax_body(b_ref, hin_ref, vp_ref, crow, cseg):
    i = pl.program_id(0)
    nblk = pl.num_programs(0)
    seg = b_ref[...]
    x = hin_ref[...]

    @pl.when(i == 0)
    def _():
        vp_ref[...] = jnp.zeros_like(vp_ref)

    nb, hh = x.shape

    pseg = jnp.where(i > 0, cseg[0:1, 0:1], -1)
    prow_ok = (i > 0)
    prow = crow[0:1, :]
    m = (seg == pseg) & prow_ok
    x = jnp.where(jnp.broadcast_to(m, (nb, hh)), jnp.maximum(x, prow), x)

    gio = lax.broadcasted_iota(jnp.int32, (_G, 1), 0)
    emit_c = (gio == pseg) & prow_ok & (pseg != seg[0:1, 0:1])
    vp_ref[...] += jnp.where(jnp.broadcast_to(emit_c, (_G, hh)),
                             jnp.broadcast_to(prow, (_G, hh)), 0.0)

    sft = 1
    while sft < nb:
        xs = jnp.concatenate(
            [jnp.full((sft, hh), _NEG, jnp.float32), x[:-sft]], axis=0)
        ss = jnp.concatenate(
            [jnp.full((sft, 1), -1, jnp.int32), seg[:-sft]], axis=0)
        x = jnp.where(jnp.broadcast_to(seg == ss, (nb, hh)),
                      jnp.maximum(x, xs), x)
        sft *= 2

    nxt = jnp.concatenate([seg[1:], jnp.full((1, 1), -2, jnp.int32)], axis=0)
    rowid = lax.broadcasted_iota(jnp.int32, (nb, 1), 0)
    endm = (((rowid != nb - 1) & (seg != nxt))
            | ((rowid == nb - 1) & (i == nblk - 1)))
    emit = jnp.where(jnp.broadcast_to(endm, (nb, hh)), x, 0.0)
    oh = (seg == lax.broadcasted_iota(jnp.int32, (1, _G), 1)
          ).astype(jnp.float32)
    vp_ref[...] += lax.dot_general(
        oh, emit, dimension_numbers=(((0,), (0,)), ((), ())),
        preferred_element_type=jnp.float32, precision=_HI)

    crow[...] = x[nb - 1:nb, :]
    cseg[...] = seg[nb - 1:nb, :]


def _segmax(b2, hin, N, H):
    nblk = N // _NB
    return pl.pallas_call(
        _segmax_body,
        grid=(nblk,),
        in_specs=[
            pl.BlockSpec((_NB, 1), lambda i: (i, 0)),
            pl.BlockSpec((_NB, H), lambda i: (i, 0)),
        ],
        out_specs=pl.BlockSpec((_G, H), lambda i: (0, 0)),
        out_shape=jax.ShapeDtypeStruct((_G, H), jnp.float32),
        scratch_shapes=[
            pltpu.VMEM((1, H), jnp.float32),
            pltpu.VMEM((1, 1), jnp.int32),
        ],
        compiler_params=pltpu.CompilerParams(
            dimension_semantics=("arbitrary",)),
    )(b2, hin)


def _vnup_body(vp_ref, vn_ref, w1_ref, b1_ref, g1_ref, be1_ref,
               w2_ref, b2_ref, out_ref):
    z = jnp.dot(vp_ref[...], w1_ref[...], preferred_element_type=jnp.float32)
    z = z + b1_ref[0:1, :]
    z = _ln(z, g1_ref[0:1, :], be1_ref[0:1, :])
    z = _gelu(z)
    z = jnp.dot(z, w2_ref[...], preferred_element_type=jnp.float32)
    z = z + b2_ref[0:1, :]
    out_ref[...] = vn_ref[...] + z


def _vnup(vp, vn, mlp, H):
    v = lambda a: a.reshape(1, -1).astype(jnp.float32)
    return pl.pallas_call(
        _vnup_body,
        out_shape=jax.ShapeDtypeStruct((_G, H), jnp.float32),
    )(vp, vn, mlp['W1'], v(mlp['b1']), v(mlp['g1']), v(mlp['be1']),
      mlp['W2'], v(mlp['b2']))


def _final_body(b_ref, h_ref, w1_ref, b1_ref, g_ref, be_ref, w2_ref, b2_ref,
                out_ref):
    i = pl.program_id(0)
    nblk = pl.num_programs(0)

    @pl.when(i == 0)
    def _():
        out_ref[...] = jnp.zeros_like(out_ref)

    seg = b_ref[...]
    oh = (seg == lax.broadcasted_iota(jnp.int32, (1, _G), 1)
          ).astype(jnp.float32)
    out_ref[...] += lax.dot_general(
        oh, h_ref[...], dimension_numbers=(((0,), (0,)), ((), ())),
        preferred_element_type=jnp.float32, precision=_HI)

    @pl.when(i == nblk - 1)
    def _():
        hg = out_ref[...]
        z = jnp.dot(hg, w1_ref[...], preferred_element_type=jnp.float32)
        z = z + b1_ref[0:1, :]
        z = _ln(z, g_ref[0:1, :], be_ref[0:1, :])
        z = _gelu(z)
        z = jnp.dot(z, w2_ref[...], preferred_element_type=jnp.float32)
        z = z + b2_ref[0:1, :]
        z = z / jnp.sqrt(jnp.sum(z * z, axis=-1, keepdims=True))
        out_ref[...] = z


def _final(b2, h, proj, N, H):
    nblk = N // _NB
    v = lambda a: a.reshape(1, -1).astype(jnp.float32)
    full = lambda shape: pl.BlockSpec(shape, lambda i: (0, 0))
    return pl.pallas_call(
        _final_body,
        grid=(nblk,),
        in_specs=[
            pl.BlockSpec((_NB, 1), lambda i: (i, 0)),
            pl.BlockSpec((_NB, H), lambda i: (i, 0)),
            full((H, H)),
            full((1, H)),
            full((1, H)),
            full((1, H)),
            full((H, H)),
            full((1, H)),
        ],
        out_specs=pl.BlockSpec((_G, H), lambda i: (0, 0)),
        out_shape=jax.ShapeDtypeStruct((_G, H), jnp.float32),
        compiler_params=pltpu.CompilerParams(
            dimension_semantics=("arbitrary",)),
    )(b2, h, proj['W1'], v(proj['b1']), v(proj['g']), v(proj['be']),
      proj['W2'], v(proj['b2']))


def kernel(x, edge_index, edge_attr, batch, params):
    N = x.shape[0]
    E = edge_index.shape[1]
    H = params['atom_enc'].shape[1]
    L = len(params['convs'])

    x2 = x.astype(jnp.int32)[:, None]
    b2 = batch.astype(jnp.int32)[:, None]
    src = edge_index[0].astype(jnp.int32)
    dst = edge_index[1].astype(jnp.int32)
    ea = edge_attr.astype(jnp.int32)

    gidx = ea * N + src
    ch = _edge_chunks(E)
    pad = 32 * _K * ch - E
    gidx2 = jnp.concatenate([gidx, jnp.full((pad,), 5 * N, jnp.int32)])
    dst2 = jnp.concatenate([dst, jnp.full((pad,), N, jnp.int32)])
    zsrc = jnp.zeros((_npad(N) // 16, H), jnp.float32)

    atom = params['atom_enc'].astype(jnp.float32)
    atom_pad = jnp.pad(atom, ((0, 128 - atom.shape[0]), (0, 0)))
    vn0 = params['vn_emb'].astype(jnp.float32)
    vn = jnp.broadcast_to(vn0[0][None, :], (_G, H)).astype(jnp.float32)

    h = None
    hin = None
    for l in range(L):
        conv = params['convs'][l]
        bond_pad = jnp.pad(conv['bond_enc'].astype(jnp.float32),
                           ((0, 3), (0, 0)))
        if l == 0:
            hin, gmsg = _pre0(x2, atom_pad, vn0, bond_pad, N, H)
        else:
            hin, gmsg = _pre(b2, h, vn, bond_pad, N, H)
        tab = jnp.concatenate(
            [gmsg.reshape(5 * N, H), jnp.zeros((8, H), jnp.float32)],
            axis=0)
        partial = _edge_agg(tab, gidx2, dst2, zsrc, N, H, E)
        h = _post(hin, partial, conv, params['norms'][l],
                  final=(l == L - 1), N=N, H=H)
        if l < L - 1:
            vp = _segmax(b2, hin, N, H)
            vn = _vnup(vp, vn, params['vn_mlps'][l], H)
    return _final(b2, h, params['proj'], N, H)


# speedup vs baseline: 2.0948x; 2.0948x over previous
import functools

import jax
import jax.numpy as jnp
from jax import lax
from jax.experimental import pallas as pl
from jax.experimental.pallas import tpu as pltpu
from jax.experimental.pallas import tpu_sc as plsc

_G = 512
_NB = 1000
_K = 128
_NEG = -1e30
_HI = jax.lax.Precision.HIGHEST


def _ln(x, g, b, eps=1e-5):
    m = jnp.mean(x, axis=-1, keepdims=True)
    v = jnp.mean((x - m) ** 2, axis=-1, keepdims=True)
    return (x - m) / jnp.sqrt(v + eps) * g + b


def _gelu(x):
    return 0.5 * x * (1.0 + lax.erf(x * (2.0 ** -0.5)))


def _pre0_body(x_ref, atom_ref, vn_ref, bond_ref, hin_ref, gmsg_ref):
    xi = x_ref[:, 0]
    oh = (xi[:, None] == lax.broadcasted_iota(jnp.int32, (1, 128), 1)
          ).astype(jnp.float32)
    h0 = jnp.dot(oh, atom_ref[...], preferred_element_type=jnp.float32,
                 precision=_HI)
    hin = h0 + vn_ref[0:1, :]
    hin_ref[...] = hin
    for a in range(5):
        gmsg_ref[a] = _gelu(hin + bond_ref[a:a + 1, :])


def _pre0(x2, atom_pad, vn0, bond_pad, N, H):
    nblk = N // _NB
    return pl.pallas_call(
        _pre0_body,
        grid=(nblk,),
        in_specs=[
            pl.BlockSpec((_NB, 1), lambda i: (i, 0)),
            pl.BlockSpec((128, H), lambda i: (0, 0)),
            pl.BlockSpec((1, H), lambda i: (0, 0)),
            pl.BlockSpec((8, H), lambda i: (0, 0)),
        ],
        out_specs=[
            pl.BlockSpec((_NB, H), lambda i: (i, 0)),
            pl.BlockSpec((5, _NB, H), lambda i: (0, i, 0)),
        ],
        out_shape=[
            jax.ShapeDtypeStruct((N, H), jnp.float32),
            jax.ShapeDtypeStruct((5, N, H), jnp.float32),
        ],
    )(x2, atom_pad, vn0, bond_pad)


def _pre_body(b_ref, hprev_ref, vn_ref, bond_ref, hin_ref, gmsg_ref):
    bi = b_ref[:, 0]
    oh = (bi[:, None] == lax.broadcasted_iota(jnp.int32, (1, _G), 1)
          ).astype(jnp.float32)
    hvn = jnp.dot(oh, vn_ref[...], preferred_element_type=jnp.float32,
                  precision=_HI)
    hin = hprev_ref[...] + hvn
    hin_ref[...] = hin
    for a in range(5):
        gmsg_ref[a] = _gelu(hin + bond_ref[a:a + 1, :])


def _pre(b2, hprev, vn, bond_pad, N, H):
    nblk = N // _NB
    return pl.pallas_call(
        _pre_body,
        grid=(nblk,),
        in_specs=[
            pl.BlockSpec((_NB, 1), lambda i: (i, 0)),
            pl.BlockSpec((_NB, H), lambda i: (i, 0)),
            pl.BlockSpec((_G, H), lambda i: (0, 0)),
            pl.BlockSpec((8, H), lambda i: (0, 0)),
        ],
        out_specs=[
            pl.BlockSpec((_NB, H), lambda i: (i, 0)),
            pl.BlockSpec((5, _NB, H), lambda i: (0, i, 0)),
        ],
        out_shape=[
            jax.ShapeDtypeStruct((N, H), jnp.float32),
            jax.ShapeDtypeStruct((5, N, H), jnp.float32),
        ],
    )(b2, hprev, vn, bond_pad)


def _npad(N):
    return 128 * ((N + 1 + 127) // 128)


def _edge_chunks(E):
    return 8 * (-(-E // (32 * _K * 8)))


def _edge_agg(tab, gidx, dst, zsrc, N, H, E):
    mesh = plsc.VectorSubcoreMesh(core_axis_name="c", subcore_axis_name="s",
                                  num_cores=2, num_subcores=16)
    npad = _npad(N)
    rpt = npad // 16
    ch = _edge_chunks(E)

    @functools.partial(
        pl.kernel,
        out_type=jax.ShapeDtypeStruct((2, npad, H), jnp.float32),
        mesh=mesh,
        scratch_types=[
            pltpu.VMEM_SHARED((npad, H), jnp.float32),
            [pltpu.VMEM((_K,), jnp.int32) for _ in range(2)],
            [pltpu.VMEM((_K,), jnp.int32) for _ in range(3)],
            [pltpu.VMEM((_K, H), jnp.float32) for _ in range(2)],
            [pltpu.SemaphoreType.DMA for _ in range(2)],
            [pltpu.SemaphoreType.DMA for _ in range(3)],
            [pltpu.SemaphoreType.DMA for _ in range(2)],
        ],
    )
    def k(tab_hbm, gidx_hbm, dst_hbm, zsrc_hbm, out_hbm,
          acc, gv, dv, rows, si, sd, sg):
        c = lax.axis_index("c")
        s = lax.axis_index("s")
        w = c * 16 + s
        base = w * ch * _K
        pltpu.sync_copy(zsrc_hbm, acc.at[pl.ds(s * rpt, rpt)])
        plsc.subcore_barrier()

        def load_idx(j):
            a = pltpu.async_copy(gidx_hbm.at[pl.ds(base + j * _K, _K)],
                                 gv[j % 2], si[j % 2])
            b = pltpu.async_copy(dst_hbm.at[pl.ds(base + j * _K, _K)],
                                 dv[j % 3], sd[j % 3])
            return a, b

        def start_gather(j, ia):
            ia.wait()
            return pltpu.async_copy(tab_hbm.at[gv[j % 2]], rows[j % 2],
                                    sg[j % 2])

        pend_i = {0: load_idx(0)}
        pend_g = {0: start_gather(0, pend_i[0][0])}
        if ch > 1:
            pend_i[1] = load_idx(1)
        for j in range(ch):
            pend_g[j].wait()
            pend_i[j][1].wait()
            if j + 1 < ch:
                pend_g[j + 1] = start_gather(j + 1, pend_i[j + 1][0])
            pltpu.sync_copy(rows[j % 2], acc.at[dv[j % 3]], add=True)
            if j + 2 < ch:
                pend_i[j + 2] = load_idx(j + 2)

        plsc.subcore_barrier()
        pltpu.sync_copy(acc.at[pl.ds(s * rpt, rpt)],
                        out_hbm.at[c, pl.ds(s * rpt, rpt)])

    return k(tab, gidx, dst, zsrc)


def _post_body(hin_ref, part_ref, w1_ref, b1_ref, g1_ref, be1_ref,
               w2_ref, b2_ref, gn_ref, bn_ref, er_ref, out_ref, *, final):
    hin = hin_ref[...]
    agg = part_ref[0].astype(jnp.float32) + part_ref[1].astype(jnp.float32)
    t = er_ref[0:1, :] * hin + agg
    z = jnp.dot(t, w1_ref[...], preferred_element_type=jnp.float32)
    z = z + b1_ref[0:1, :]
    z = _ln(z, g1_ref[0:1, :], be1_ref[0:1, :])
    z = _gelu(z)
    z = jnp.dot(z, w2_ref[...], preferred_element_type=jnp.float32)
    z = z + b2_ref[0:1, :]
    z = _ln(z, gn_ref[0:1, :], bn_ref[0:1, :])
    if not final:
        z = _gelu(z)
    out_ref[...] = z + hin


def _post(hin, partial, conv, norm, final, N, H):
    nblk = N // _NB
    er = jnp.ones((1, H), jnp.float32) * (1.0 + conv['eps'])
    v = lambda a: a.reshape(1, -1).astype(jnp.float32)
    full = lambda shape: pl.BlockSpec(shape, lambda i: (0, 0))
    return pl.pallas_call(
        functools.partial(_post_body, final=final),
        grid=(nblk,),
        in_specs=[
            pl.BlockSpec((_NB, H), lambda i: (i, 0)),
            pl.BlockSpec((2, _NB, H), lambda i: (0, i, 0)),
            full((H, 4 * H)),
            full((1, 4 * H)),
            full((1, 4 * H)),
            full((1, 4 * H)),
            full((4 * H, H)),
            full((1, H)),
            full((1, H)),
            full((1, H)),
            full((1, H)),
        ],
        out_specs=pl.BlockSpec((_NB, H), lambda i: (i, 0)),
        out_shape=jax.ShapeDtypeStruct((N, H), jnp.float32),
    )(hin, partial, conv['W1'], v(conv['b1']), v(conv['g1']), v(conv['be1']),
      conv['W2'], v(conv['b2']), v(norm['g']), v(norm['b']), er)


def _segmax_body(b_ref, hin_ref, vp_ref, crow, cseg):
    i = pl.program_id(0)
    nblk = pl.num_programs(0)
    seg = b_ref[...]
    x = hin_ref[...]

    @pl.when(i == 0)
    def _():
        vp_ref[...] = jnp.zeros_like(vp_ref)

    nb, hh = x.shape

    pseg = jnp.where(i > 0, cseg[0:1, 0:1], -1)
    prow_ok = (i > 0)
    prow = crow[0:1, :]
    m = (seg == pseg) & prow_ok
    x = jnp.where(jnp.broadcast_to(m, (nb, hh)), jnp.maximum(x, prow), x)

    gio = lax.broadcasted_iota(jnp.int32, (_G, 1), 0)
    emit_c = (gio == pseg) & prow_ok & (pseg != seg[0:1, 0:1])
    vp_ref[...] += jnp.where(jnp.broadcast_to(emit_c, (_G, hh)),
                             jnp.broadcast_to(prow, (_G, hh)), 0.0)

    sft = 1
    while sft < nb:
        xs = jnp.concatenate(
            [jnp.full((sft, hh), _NEG, jnp.float32), x[:-sft]], axis=0)
        ss = jnp.concatenate(
            [jnp.full((sft, 1), -1, jnp.int32), seg[:-sft]], axis=0)
        x = jnp.where(jnp.broadcast_to(seg == ss, (nb, hh)),
                      jnp.maximum(x, xs), x)
        sft *= 2

    nxt = jnp.concatenate([seg[1:], jnp.full((1, 1), -2, jnp.int32)], axis=0)
    rowid = lax.broadcasted_iota(jnp.int32, (nb, 1), 0)
    endm = (((rowid != nb - 1) & (seg != nxt))
            | ((rowid == nb - 1) & (i == nblk - 1)))
    emit = jnp.where(jnp.broadcast_to(endm, (nb, hh)), x, 0.0)
    oh = (seg == lax.broadcasted_iota(jnp.int32, (1, _G), 1)
          ).astype(jnp.float32)
    vp_ref[...] += lax.dot_general(
        oh, emit, dimension_numbers=(((0,), (0,)), ((), ())),
        preferred_element_type=jnp.float32, precision=_HI)

    crow[...] = x[nb - 1:nb, :]
    cseg[...] = seg[nb - 1:nb, :]


def _segmax(b2, hin, N, H):
    nblk = N // _NB
    return pl.pallas_call(
        _segmax_body,
        grid=(nblk,),
        in_specs=[
            pl.BlockSpec((_NB, 1), lambda i: (i, 0)),
            pl.BlockSpec((_NB, H), lambda i: (i, 0)),
        ],
        out_specs=pl.BlockSpec((_G, H), lambda i: (0, 0)),
        out_shape=jax.ShapeDtypeStruct((_G, H), jnp.float32),
        scratch_shapes=[
            pltpu.VMEM((1, H), jnp.float32),
            pltpu.VMEM((1, 1), jnp.int32),
        ],
        compiler_params=pltpu.CompilerParams(
            dimension_semantics=("arbitrary",)),
    )(b2, hin)


def _vnup_body(vp_ref, vn_ref, w1_ref, b1_ref, g1_ref, be1_ref,
               w2_ref, b2_ref, out_ref):
    z = jnp.dot(vp_ref[...], w1_ref[...], preferred_element_type=jnp.float32)
    z = z + b1_ref[0:1, :]
    z = _ln(z, g1_ref[0:1, :], be1_ref[0:1, :])
    z = _gelu(z)
    z = jnp.dot(z, w2_ref[...], preferred_element_type=jnp.float32)
    z = z + b2_ref[0:1, :]
    out_ref[...] = vn_ref[...] + z


def _vnup(vp, vn, mlp, H):
    v = lambda a: a.reshape(1, -1).astype(jnp.float32)
    return pl.pallas_call(
        _vnup_body,
        out_shape=jax.ShapeDtypeStruct((_G, H), jnp.float32),
    )(vp, vn, mlp['W1'], v(mlp['b1']), v(mlp['g1']), v(mlp['be1']),
      mlp['W2'], v(mlp['b2']))


def _final_body(b_ref, h_ref, w1_ref, b1_ref, g_ref, be_ref, w2_ref, b2_ref,
                out_ref):
    i = pl.program_id(0)
    nblk = pl.num_programs(0)

    @pl.when(i == 0)
    def _():
        out_ref[...] = jnp.zeros_like(out_ref)

    seg = b_ref[...]
    oh = (seg == lax.broadcasted_iota(jnp.int32, (1, _G), 1)
          ).astype(jnp.float32)
    out_ref[...] += lax.dot_general(
        oh, h_ref[...], dimension_numbers=(((0,), (0,)), ((), ())),
        preferred_element_type=jnp.float32, precision=_HI)

    @pl.when(i == nblk - 1)
    def _():
        hg = out_ref[...]
        z = jnp.dot(hg, w1_ref[...], preferred_element_type=jnp.float32)
        z = z + b1_ref[0:1, :]
        z = _ln(z, g_ref[0:1, :], be_ref[0:1, :])
        z = _gelu(z)
        z = jnp.dot(z, w2_ref[...], preferred_element_type=jnp.float32)
        z = z + b2_ref[0:1, :]
        z = z / jnp.sqrt(jnp.sum(z * z, axis=-1, keepdims=True))
        out_ref[...] = z


def _final(b2, h, proj, N, H):
    nblk = N // _NB
    v = lambda a: a.reshape(1, -1).astype(jnp.float32)
    full = lambda shape: pl.BlockSpec(shape, lambda i: (0, 0))
    return pl.pallas_call(
        _final_body,
        grid=(nblk,),
        in_specs=[
            pl.BlockSpec((_NB, 1), lambda i: (i, 0)),
            pl.BlockSpec((_NB, H), lambda i: (i, 0)),
            full((H, H)),
            full((1, H)),
            full((1, H)),
            full((1, H)),
            full((H, H)),
            full((1, H)),
        ],
        out_specs=pl.BlockSpec((_G, H), lambda i: (0, 0)),
        out_shape=jax.ShapeDtypeStruct((_G, H), jnp.float32),
        compiler_params=pltpu.CompilerParams(
            dimension_semantics=("arbitrary",)),
    )(b2, h, proj['W1'], v(proj['b1']), v(proj['g']), v(proj['be']),
      proj['W2'], v(proj['b2']))


def kernel(x, edge_index, edge_attr, batch, params):
    N = x.shape[0]
    E = edge_index.shape[1]
    H = params['atom_enc'].shape[1]
    L = len(params['convs'])

    x2 = x.astype(jnp.int32)[:, None]
    b2 = batch.astype(jnp.int32)[:, None]
    src = edge_index[0].astype(jnp.int32)
    dst = edge_index[1].astype(jnp.int32)
    ea = edge_attr.astype(jnp.int32)

    gidx = ea * N + src
    ch = _edge_chunks(E)
    pad = 32 * _K * ch - E
    gidx2 = jnp.concatenate([gidx, jnp.full((pad,), 5 * N, jnp.int32)])
    dst2 = jnp.concatenate([dst, jnp.full((pad,), N, jnp.int32)])
    zsrc = jnp.zeros((_npad(N) // 16, H), jnp.float32)

    atom = params['atom_enc'].astype(jnp.float32)
    atom_pad = jnp.pad(atom, ((0, 128 - atom.shape[0]), (0, 0)))
    vn0 = params['vn_emb'].astype(jnp.float32)
    vn = jnp.broadcast_to(vn0[0][None, :], (_G, H)).astype(jnp.float32)

    h = None
    hin = None
    for l in range(L):
        conv = params['convs'][l]
        bond_pad = jnp.pad(conv['bond_enc'].astype(jnp.float32),
                           ((0, 3), (0, 0)))
        if l == 0:
            hin, gmsg = _pre0(x2, atom_pad, vn0, bond_pad, N, H)
        else:
            hin, gmsg = _pre(b2, h, vn, bond_pad, N, H)
        tab = jnp.concatenate(
            [gmsg.reshape(5 * N, H), jnp.zeros((8, H), jnp.float32)],
            axis=0)
        partial = _edge_agg(tab, gidx2, dst2, zsrc, N, H, E)
        h = _post(hin, partial, conv, params['norms'][l],
                  final=(l == L - 1), N=N, H=H)
        if l < L - 1:
            vp = _segmax(b2, hin, N, H)
            vn = _vnup(vp, vn, params['vn_mlps'][l], H)
    return _final(b2, h, params['proj'], N, H)


# speedup vs baseline: 2.0957x; 1.0005x over previous
import functools

import jax
import jax.numpy as jnp
from jax import lax
from jax.experimental import pallas as pl
from jax.experimental.pallas import tpu as pltpu
from jax.experimental.pallas import tpu_sc as plsc

_G = 512
_NB = 1000
_K = 128
_NEG = -1e30
_HI = jax.lax.Precision.HIGHEST


def _ln(x, g, b, eps=1e-5):
    m = jnp.mean(x, axis=-1, keepdims=True)
    v = jnp.mean((x - m) ** 2, axis=-1, keepdims=True)
    return (x - m) / jnp.sqrt(v + eps) * g + b


def _gelu(x):
    return 0.5 * x * (1.0 + lax.erf(x * (2.0 ** -0.5)))


def _pre0_body(x_ref, atom_ref, vn_ref, bond_ref, hin_ref, gmsg_ref):
    xi = x_ref[:, 0]
    oh = (xi[:, None] == lax.broadcasted_iota(jnp.int32, (1, 128), 1)
          ).astype(jnp.float32)
    h0 = jnp.dot(oh, atom_ref[...], preferred_element_type=jnp.float32,
                 precision=_HI)
    hin = h0 + vn_ref[0:1, :]
    hin_ref[...] = hin
    for a in range(5):
        gmsg_ref[a] = _gelu(hin + bond_ref[a:a + 1, :])


def _pre0(x2, atom_pad, vn0, bond_pad, N, H):
    nblk = N // _NB
    return pl.pallas_call(
        _pre0_body,
        grid=(nblk,),
        in_specs=[
            pl.BlockSpec((_NB, 1), lambda i: (i, 0)),
            pl.BlockSpec((128, H), lambda i: (0, 0)),
            pl.BlockSpec((1, H), lambda i: (0, 0)),
            pl.BlockSpec((8, H), lambda i: (0, 0)),
        ],
        out_specs=[
            pl.BlockSpec((_NB, H), lambda i: (i, 0)),
            pl.BlockSpec((5, _NB, H), lambda i: (0, i, 0)),
        ],
        out_shape=[
            jax.ShapeDtypeStruct((N, H), jnp.float32),
            jax.ShapeDtypeStruct((5, N, H), jnp.float32),
        ],
    )(x2, atom_pad, vn0, bond_pad)


def _pre_body(b_ref, hprev_ref, vn_ref, bond_ref, hin_ref, gmsg_ref):
    bi = b_ref[:, 0]
    oh = (bi[:, None] == lax.broadcasted_iota(jnp.int32, (1, _G), 1)
          ).astype(jnp.float32)
    hvn = jnp.dot(oh, vn_ref[...], preferred_element_type=jnp.float32,
                  precision=_HI)
    hin = hprev_ref[...] + hvn
    hin_ref[...] = hin
    for a in range(5):
        gmsg_ref[a] = _gelu(hin + bond_ref[a:a + 1, :])


def _pre(b2, hprev, vn, bond_pad, N, H):
    nblk = N // _NB
    return pl.pallas_call(
        _pre_body,
        grid=(nblk,),
        in_specs=[
            pl.BlockSpec((_NB, 1), lambda i: (i, 0)),
            pl.BlockSpec((_NB, H), lambda i: (i, 0)),
            pl.BlockSpec((_G, H), lambda i: (0, 0)),
            pl.BlockSpec((8, H), lambda i: (0, 0)),
        ],
        out_specs=[
            pl.BlockSpec((_NB, H), lambda i: (i, 0)),
            pl.BlockSpec((5, _NB, H), lambda i: (0, i, 0)),
        ],
        out_shape=[
            jax.ShapeDtypeStruct((N, H), jnp.float32),
            jax.ShapeDtypeStruct((5, N, H), jnp.float32),
        ],
    )(b2, hprev, vn, bond_pad)


def _npad(N):
    return 128 * ((N + 1 + 127) // 128)


def _edge_chunks(E):
    return 8 * (-(-E // (32 * _K * 8)))


def _edge_agg(tab, gidx, dst, zsrc, N, H, E):
    mesh = plsc.VectorSubcoreMesh(core_axis_name="c", subcore_axis_name="s",
                                  num_cores=2, num_subcores=16)
    npad = _npad(N)
    rpt = npad // 16
    ch = _edge_chunks(E)

    @functools.partial(
        pl.kernel,
        out_type=jax.ShapeDtypeStruct((2, npad, H), jnp.float32),
        mesh=mesh,
        scratch_types=[
            pltpu.VMEM_SHARED((npad, H), jnp.float32),
            [pltpu.VMEM((_K,), jnp.int32) for _ in range(2)],
            [pltpu.VMEM((_K,), jnp.int32) for _ in range(3)],
            [pltpu.VMEM((_K, H), jnp.float32) for _ in range(2)],
            [pltpu.SemaphoreType.DMA for _ in range(2)],
            [pltpu.SemaphoreType.DMA for _ in range(3)],
            [pltpu.SemaphoreType.DMA for _ in range(2)],
        ],
    )
    def k(tab_hbm, gidx_hbm, dst_hbm, zsrc_hbm, out_hbm,
          acc, gv, dv, rows, si, sd, sg):
        c = lax.axis_index("c")
        s = lax.axis_index("s")
        w = c * 16 + s
        base = w * ch * _K
        pltpu.sync_copy(zsrc_hbm, acc.at[pl.ds(s * rpt, rpt)])
        plsc.subcore_barrier()

        def load_idx(j):
            a = pltpu.async_copy(gidx_hbm.at[pl.ds(base + j * _K, _K)],
                                 gv[j % 2], si[j % 2])
            b = pltpu.async_copy(dst_hbm.at[pl.ds(base + j * _K, _K)],
                                 dv[j % 3], sd[j % 3])
            return a, b

        def start_gather(j, ia):
            ia.wait()
            return pltpu.async_copy(tab_hbm.at[gv[j % 2]], rows[j % 2],
                                    sg[j % 2])

        pend_i = {0: load_idx(0)}
        pend_g = {0: start_gather(0, pend_i[0][0])}
        if ch > 1:
            pend_i[1] = load_idx(1)
        for j in range(ch):
            pend_g[j].wait()
            pend_i[j][1].wait()
            if j + 1 < ch:
                pend_g[j + 1] = start_gather(j + 1, pend_i[j + 1][0])
            pltpu.sync_copy(rows[j % 2], acc.at[dv[j % 3]], add=True)
            if j + 2 < ch:
                pend_i[j + 2] = load_idx(j + 2)

        plsc.subcore_barrier()
        pltpu.sync_copy(acc.at[pl.ds(s * rpt, rpt)],
                        out_hbm.at[c, pl.ds(s * rpt, rpt)])

    return k(tab, gidx, dst, zsrc)


def _post_body(hin_ref, part_ref, w1_ref, b1_ref, g1_ref, be1_ref,
               w2_ref, b2_ref, gn_ref, bn_ref, er_ref, out_ref, *, final):
    hin = hin_ref[...]
    agg = part_ref[0] + part_ref[1]
    t = er_ref[0:1, :] * hin + agg
    z = jnp.dot(t, w1_ref[...], preferred_element_type=jnp.float32)
    z = z + b1_ref[0:1, :]
    z = _ln(z, g1_ref[0:1, :], be1_ref[0:1, :])
    z = _gelu(z)
    z = jnp.dot(z, w2_ref[...], preferred_element_type=jnp.float32)
    z = z + b2_ref[0:1, :]
    z = _ln(z, gn_ref[0:1, :], bn_ref[0:1, :])
    if not final:
        z = _gelu(z)
    out_ref[...] = z + hin


def _post(hin, partial, conv, norm, final, N, H):
    nblk = N // _NB
    er = jnp.ones((1, H), jnp.float32) * (1.0 + conv['eps'])
    v = lambda a: a.reshape(1, -1).astype(jnp.float32)
    full = lambda shape: pl.BlockSpec(shape, lambda i: (0, 0))
    return pl.pallas_call(
        functools.partial(_post_body, final=final),
        grid=(nblk,),
        in_specs=[
            pl.BlockSpec((_NB, H), lambda i: (i, 0)),
            pl.BlockSpec((2, _NB, H), lambda i: (0, i, 0)),
            full((H, 4 * H)),
            full((1, 4 * H)),
            full((1, 4 * H)),
            full((1, 4 * H)),
            full((4 * H, H)),
            full((1, H)),
            full((1, H)),
            full((1, H)),
            full((1, H)),
        ],
        out_specs=pl.BlockSpec((_NB, H), lambda i: (i, 0)),
        out_shape=jax.ShapeDtypeStruct((N, H), jnp.float32),
    )(hin, partial, conv['W1'], v(conv['b1']), v(conv['g1']), v(conv['be1']),
      conv['W2'], v(conv['b2']), v(norm['g']), v(norm['b']), er)


def _segmax_body(b_ref, hin_ref, vp_ref, crow, cseg):
    i = pl.program_id(0)
    nblk = pl.num_programs(0)
    seg = b_ref[...]
    x = hin_ref[...]

    @pl.when(i == 0)
    def _():
        vp_ref[...] = jnp.zeros_like(vp_ref)

    nb, hh = x.shape

    pseg = jnp.where(i > 0, cseg[0:1, 0:1], -1)
    prow_ok = (i > 0)
    prow = crow[0:1, :]
    m = (seg == pseg) & prow_ok
    x = jnp.where(jnp.broadcast_to(m, (nb, hh)), jnp.maximum(x, prow), x)

    gio = lax.broadcasted_iota(jnp.int32, (_G, 1), 0)
    emit_c = (gio == pseg) & prow_ok & (pseg != seg[0:1, 0:1])
    vp_ref[...] += jnp.where(jnp.broadcast_to(emit_c, (_G, hh)),
                             jnp.broadcast_to(prow, (_G, hh)), 0.0)

    sft = 1
    while sft < nb:
        xs = jnp.concatenate(
            [jnp.full((sft, hh), _NEG, jnp.float32), x[:-sft]], axis=0)
        ss = jnp.concatenate(
            [jnp.full((sft, 1), -1, jnp.int32), seg[:-sft]], axis=0)
        x = jnp.where(jnp.broadcast_to(seg == ss, (nb, hh)),
                      jnp.maximum(x, xs), x)
        sft *= 2

    nxt = jnp.concatenate([seg[1:], jnp.full((1, 1), -2, jnp.int32)], axis=0)
    rowid = lax.broadcasted_iota(jnp.int32, (nb, 1), 0)
    endm = (((rowid != nb - 1) & (seg != nxt))
            | ((rowid == nb - 1) & (i == nblk - 1)))
    emit = jnp.where(jnp.broadcast_to(endm, (nb, hh)), x, 0.0)
    oh = (seg == lax.broadcasted_iota(jnp.int32, (1, _G), 1)
          ).astype(jnp.float32)
    vp_ref[...] += lax.dot_general(
        oh, emit, dimension_numbers=(((0,), (0,)), ((), ())),
        preferred_element_type=jnp.float32, precision=_HI)

    crow[...] = x[nb - 1:nb, :]
    cseg[...] = seg[nb - 1:nb, :]


def _segmax(b2, hin, N, H):
    nblk = N // _NB
    return pl.pallas_call(
        _segmax_body,
        grid=(nblk,),
        in_specs=[
            pl.BlockSpec((_NB, 1), lambda i: (i, 0)),
            pl.BlockSpec((_NB, H), lambda i: (i, 0)),
        ],
        out_specs=pl.BlockSpec((_G, H), lambda i: (0, 0)),
        out_shape=jax.ShapeDtypeStruct((_G, H), jnp.float32),
        scratch_shapes=[
            pltpu.VMEM((1, H), jnp.float32),
            pltpu.VMEM((1, 1), jnp.int32),
        ],
        compiler_params=pltpu.CompilerParams(
            dimension_semantics=("arbitrary",)),
    )(b2, hin)


def _vnup_body(vp_ref, vn_ref, w1_ref, b1_ref, g1_ref, be1_ref,
               w2_ref, b2_ref, out_ref):
    z = jnp.dot(vp_ref[...], w1_ref[...], preferred_element_type=jnp.float32)
    z = z + b1_ref[0:1, :]
    z = _ln(z, g1_ref[0:1, :], be1_ref[0:1, :])
    z = _gelu(z)
    z = jnp.dot(z, w2_ref[...], preferred_element_type=jnp.float32)
    z = z + b2_ref[0:1, :]
    out_ref[...] = vn_ref[...] + z


def _vnup(vp, vn, mlp, H):
    v = lambda a: a.reshape(1, -1).astype(jnp.float32)
    return pl.pallas_call(
        _vnup_body,
        out_shape=jax.ShapeDtypeStruct((_G, H), jnp.float32),
    )(vp, vn, mlp['W1'], v(mlp['b1']), v(mlp['g1']), v(mlp['be1']),
      mlp['W2'], v(mlp['b2']))


def _final_body(b_ref, h_ref, w1_ref, b1_ref, g_ref, be_ref, w2_ref, b2_ref,
                out_ref):
    i = pl.program_id(0)
    nblk = pl.num_programs(0)

    @pl.when(i == 0)
    def _():
        out_ref[...] = jnp.zeros_like(out_ref)

    seg = b_ref[...]
    oh = (seg == lax.broadcasted_iota(jnp.int32, (1, _G), 1)
          ).astype(jnp.float32)
    out_ref[...] += lax.dot_general(
        oh, h_ref[...], dimension_numbers=(((0,), (0,)), ((), ())),
        preferred_element_type=jnp.float32, precision=_HI)

    @pl.when(i == nblk - 1)
    def _():
        hg = out_ref[...]
        z = jnp.dot(hg, w1_ref[...], preferred_element_type=jnp.float32)
        z = z + b1_ref[0:1, :]
        z = _ln(z, g_ref[0:1, :], be_ref[0:1, :])
        z = _gelu(z)
        z = jnp.dot(z, w2_ref[...], preferred_element_type=jnp.float32)
        z = z + b2_ref[0:1, :]
        z = z / jnp.sqrt(jnp.sum(z * z, axis=-1, keepdims=True))
        out_ref[...] = z


def _final(b2, h, proj, N, H):
    nblk = N // _NB
    v = lambda a: a.reshape(1, -1).astype(jnp.float32)
    full = lambda shape: pl.BlockSpec(shape, lambda i: (0, 0))
    return pl.pallas_call(
        _final_body,
        grid=(nblk,),
        in_specs=[
            pl.BlockSpec((_NB, 1), lambda i: (i, 0)),
            pl.BlockSpec((_NB, H), lambda i: (i, 0)),
            full((H, H)),
            full((1, H)),
            full((1, H)),
            full((1, H)),
            full((H, H)),
            full((1, H)),
        ],
        out_specs=pl.BlockSpec((_G, H), lambda i: (0, 0)),
        out_shape=jax.ShapeDtypeStruct((_G, H), jnp.float32),
        compiler_params=pltpu.CompilerParams(
            dimension_semantics=("arbitrary",)),
    )(b2, h, proj['W1'], v(proj['b1']), v(proj['g']), v(proj['be']),
      proj['W2'], v(proj['b2']))


def kernel(x, edge_index, edge_attr, batch, params):
    N = x.shape[0]
    E = edge_index.shape[1]
    H = params['atom_enc'].shape[1]
    L = len(params['convs'])

    x2 = x.astype(jnp.int32)[:, None]
    b2 = batch.astype(jnp.int32)[:, None]
    src = edge_index[0].astype(jnp.int32)
    dst = edge_index[1].astype(jnp.int32)
    ea = edge_attr.astype(jnp.int32)

    gidx = ea * N + src
    ch = _edge_chunks(E)
    pad = 32 * _K * ch - E
    gidx2 = jnp.concatenate([gidx, jnp.full((pad,), 5 * N, jnp.int32)])
    dst2 = jnp.concatenate([dst, jnp.full((pad,), N, jnp.int32)])
    zsrc = jnp.zeros((_npad(N) // 16, H), jnp.float32)

    atom = params['atom_enc'].astype(jnp.float32)
    atom_pad = jnp.pad(atom, ((0, 128 - atom.shape[0]), (0, 0)))
    vn0 = params['vn_emb'].astype(jnp.float32)
    vn = jnp.broadcast_to(vn0[0][None, :], (_G, H)).astype(jnp.float32)

    h = None
    hin = None
    for l in range(L):
        conv = params['convs'][l]
        bond_pad = jnp.pad(conv['bond_enc'].astype(jnp.float32),
                           ((0, 3), (0, 0)))
        if l == 0:
            hin, gmsg = _pre0(x2, atom_pad, vn0, bond_pad, N, H)
        else:
            hin, gmsg = _pre(b2, h, vn, bond_pad, N, H)
        tab = jnp.concatenate(
            [gmsg.reshape(5 * N, H), jnp.zeros((8, H), jnp.float32)],
            axis=0)
        partial = _edge_agg(tab, gidx2, dst2, zsrc, N, H, E)
        h = _post(hin, partial, conv, params['norms'][l],
                  final=(l == L - 1), N=N, H=H)
        if l < L - 1:
            vp = _segmax(b2, hin, N, H)
            vn = _vnup(vp, vn, params['vn_mlps'][l], H)
    return _final(b2, h, params['proj'], N, H)


# speedup vs baseline: 5.1784x; 2.4709x over previous
import functools

import jax
import jax.numpy as jnp
from jax import lax
from jax.experimental import pallas as pl
from jax.experimental.pallas import tpu as pltpu
from jax.experimental.pallas import tpu_sc as plsc

_G = 512
_NB = 1000
_K = 128
_NEG = -1e30
_HI = jax.lax.Precision.HIGHEST


def _ln(x, g, b, eps=1e-5):
    m = jnp.mean(x, axis=-1, keepdims=True)
    v = jnp.mean((x - m) ** 2, axis=-1, keepdims=True)
    return (x - m) / jnp.sqrt(v + eps) * g + b


def _gelu(x):
    return 0.5 * x * (1.0 + lax.erf(x * (2.0 ** -0.5)))


def _pre0_body(x_ref, atom_ref, vn_ref, bond_ref, hin_ref, gmsg_ref):
    xi = x_ref[:, 0]
    oh = (xi[:, None] == lax.broadcasted_iota(jnp.int32, (1, 128), 1)
          ).astype(jnp.float32)
    h0 = jnp.dot(oh, atom_ref[...], preferred_element_type=jnp.float32,
                 precision=_HI)
    hin = h0 + vn_ref[0:1, :]
    hin_ref[...] = hin
    for a in range(5):
        gmsg_ref[a] = _gelu(hin + bond_ref[a:a + 1, :])


def _pre0(x2, atom_pad, vn0, bond_pad, N, H):
    nblk = N // _NB
    return pl.pallas_call(
        _pre0_body,
        grid=(nblk,),
        in_specs=[
            pl.BlockSpec((_NB, 1), lambda i: (i, 0)),
            pl.BlockSpec((128, H), lambda i: (0, 0)),
            pl.BlockSpec((1, H), lambda i: (0, 0)),
            pl.BlockSpec((8, H), lambda i: (0, 0)),
        ],
        out_specs=[
            pl.BlockSpec((_NB, H), lambda i: (i, 0)),
            pl.BlockSpec((5, _NB, H), lambda i: (0, i, 0)),
        ],
        out_shape=[
            jax.ShapeDtypeStruct((N, H), jnp.float32),
            jax.ShapeDtypeStruct((5, N, H), jnp.float32),
        ],
    )(x2, atom_pad, vn0, bond_pad)


def _pre_body(b_ref, hprev_ref, vn_ref, bond_ref, hin_ref, gmsg_ref):
    bi = b_ref[:, 0]
    oh = (bi[:, None] == lax.broadcasted_iota(jnp.int32, (1, _G), 1)
          ).astype(jnp.float32)
    hvn = jnp.dot(oh, vn_ref[...], preferred_element_type=jnp.float32,
                  precision=_HI)
    hin = hprev_ref[...] + hvn
    hin_ref[...] = hin
    for a in range(5):
        gmsg_ref[a] = _gelu(hin + bond_ref[a:a + 1, :])


def _pre(b2, hprev, vn, bond_pad, N, H):
    nblk = N // _NB
    return pl.pallas_call(
        _pre_body,
        grid=(nblk,),
        in_specs=[
            pl.BlockSpec((_NB, 1), lambda i: (i, 0)),
            pl.BlockSpec((_NB, H), lambda i: (i, 0)),
            pl.BlockSpec((_G, H), lambda i: (0, 0)),
            pl.BlockSpec((8, H), lambda i: (0, 0)),
        ],
        out_specs=[
            pl.BlockSpec((_NB, H), lambda i: (i, 0)),
            pl.BlockSpec((5, _NB, H), lambda i: (0, i, 0)),
        ],
        out_shape=[
            jax.ShapeDtypeStruct((N, H), jnp.float32),
            jax.ShapeDtypeStruct((5, N, H), jnp.float32),
        ],
    )(b2, hprev, vn, bond_pad)


def _npad(N):
    return 128 * ((N + 1 + 127) // 128)


def _edge_chunks(E):
    return 8 * (-(-E // (32 * _K * 8)))


def _edge_agg(tab, gidx, dst, zsrc, N, H, E):
    mesh = plsc.VectorSubcoreMesh(core_axis_name="c", subcore_axis_name="s",
                                  num_cores=2, num_subcores=16)
    npad = _npad(N)
    rpt = npad // 16
    ch = _edge_chunks(E)

    @functools.partial(
        pl.kernel,
        out_type=jax.ShapeDtypeStruct((2, npad, H), jnp.float32),
        mesh=mesh,
        scratch_types=[
            pltpu.VMEM_SHARED((npad, H), jnp.float32),
            [pltpu.VMEM((_K,), jnp.int32) for _ in range(2)],
            [pltpu.VMEM((_K,), jnp.int32) for _ in range(3)],
            [pltpu.VMEM((_K, H), jnp.float32) for _ in range(2)],
            [pltpu.SemaphoreType.DMA for _ in range(2)],
            [pltpu.SemaphoreType.DMA for _ in range(3)],
            [pltpu.SemaphoreType.DMA for _ in range(2)],
        ],
    )
    def k(tab_hbm, gidx_hbm, dst_hbm, zsrc_hbm, out_hbm,
          acc, gv, dv, rows, si, sd, sg):
        c = lax.axis_index("c")
        s = lax.axis_index("s")
        w = c * 16 + s
        base = w * ch * _K
        pltpu.sync_copy(zsrc_hbm, acc.at[pl.ds(s * rpt, rpt)])
        plsc.subcore_barrier()

        def load_idx(j):
            a = pltpu.async_copy(gidx_hbm.at[pl.ds(base + j * _K, _K)],
                                 gv[j % 2], si[j % 2])
            b = pltpu.async_copy(dst_hbm.at[pl.ds(base + j * _K, _K)],
                                 dv[j % 3], sd[j % 3])
            return a, b

        def start_gather(j, ia):
            ia.wait()
            return pltpu.async_copy(tab_hbm.at[gv[j % 2]], rows[j % 2],
                                    sg[j % 2])

        pend_i = {0: load_idx(0)}
        pend_g = {0: start_gather(0, pend_i[0][0])}
        if ch > 1:
            pend_i[1] = load_idx(1)
        for j in range(ch):
            pend_g[j].wait()
            pend_i[j][1].wait()
            if j + 1 < ch:
                pend_g[j + 1] = start_gather(j + 1, pend_i[j + 1][0])
            pltpu.sync_copy(rows[j % 2], acc.at[dv[j % 3]], add=True)
            if j + 2 < ch:
                pend_i[j + 2] = load_idx(j + 2)

        plsc.subcore_barrier()
        pltpu.sync_copy(acc.at[pl.ds(s * rpt, rpt)],
                        out_hbm.at[c, pl.ds(s * rpt, rpt)])

    return k(tab, gidx, dst, zsrc)


def _post_body(hin_ref, part_ref, w1_ref, b1_ref, g1_ref, be1_ref,
               w2_ref, b2_ref, gn_ref, bn_ref, er_ref, out_ref, *, final):
    hin = hin_ref[...]
    agg = part_ref[0] + part_ref[1]
    t = er_ref[0:1, :] * hin + agg
    z = jnp.dot(t, w1_ref[...], preferred_element_type=jnp.float32)
    z = z + b1_ref[0:1, :]
    z = _ln(z, g1_ref[0:1, :], be1_ref[0:1, :])
    z = _gelu(z)
    z = jnp.dot(z, w2_ref[...], preferred_element_type=jnp.float32)
    z = z + b2_ref[0:1, :]
    z = _ln(z, gn_ref[0:1, :], bn_ref[0:1, :])
    if not final:
        z = _gelu(z)
    out_ref[...] = z + hin


def _post(hin, partial, conv, norm, final, N, H):
    nblk = N // _NB
    er = jnp.ones((1, H), jnp.float32) * (1.0 + conv['eps'])
    v = lambda a: a.reshape(1, -1).astype(jnp.float32)
    full = lambda shape: pl.BlockSpec(shape, lambda i: (0, 0))
    return pl.pallas_call(
        functools.partial(_post_body, final=final),
        grid=(nblk,),
        in_specs=[
            pl.BlockSpec((_NB, H), lambda i: (i, 0)),
            pl.BlockSpec((2, _NB, H), lambda i: (0, i, 0)),
            full((H, 4 * H)),
            full((1, 4 * H)),
            full((1, 4 * H)),
            full((1, 4 * H)),
            full((4 * H, H)),
            full((1, H)),
            full((1, H)),
            full((1, H)),
            full((1, H)),
        ],
        out_specs=pl.BlockSpec((_NB, H), lambda i: (i, 0)),
        out_shape=jax.ShapeDtypeStruct((N, H), jnp.float32),
    )(hin, partial, conv['W1'], v(conv['b1']), v(conv['g1']), v(conv['be1']),
      conv['W2'], v(conv['b2']), v(norm['g']), v(norm['b']), er)


def _segmax_body(b_ref, hin_ref, vp_ref, crow, cseg):
    i = pl.program_id(0)
    nblk = pl.num_programs(0)
    seg = b_ref[...]
    x = hin_ref[...]

    @pl.when(i == 0)
    def _():
        vp_ref[...] = jnp.zeros_like(vp_ref)

    nb, hh = x.shape

    pseg = jnp.where(i > 0, cseg[0:1, 0:1], -1)
    prow_ok = (i > 0)
    prow = crow[0:1, :]
    m = (seg == pseg) & prow_ok
    x = jnp.where(jnp.broadcast_to(m, (nb, hh)), jnp.maximum(x, prow), x)

    gio = lax.broadcasted_iota(jnp.int32, (_G, 1), 0)
    emit_c = (gio == pseg) & prow_ok & (pseg != seg[0:1, 0:1])
    vp_ref[...] += jnp.where(jnp.broadcast_to(emit_c, (_G, hh)),
                             jnp.broadcast_to(prow, (_G, hh)), 0.0)

    sft = 1
    while sft < nb:
        xs = jnp.concatenate(
            [jnp.full((sft, hh), _NEG, jnp.float32), x[:-sft]], axis=0)
        ss = jnp.concatenate(
            [jnp.full((sft, 1), -1, jnp.int32), seg[:-sft]], axis=0)
        x = jnp.where(jnp.broadcast_to(seg == ss, (nb, hh)),
                      jnp.maximum(x, xs), x)
        sft *= 2

    nxt = jnp.concatenate([seg[1:], jnp.full((1, 1), -2, jnp.int32)], axis=0)
    rowid = lax.broadcasted_iota(jnp.int32, (nb, 1), 0)
    endm = (((rowid != nb - 1) & (seg != nxt))
            | ((rowid == nb - 1) & (i == nblk - 1)))
    emit = jnp.where(jnp.broadcast_to(endm, (nb, hh)), x, 0.0)
    oh = (seg == lax.broadcasted_iota(jnp.int32, (1, _G), 1)
          ).astype(jnp.float32)
    vp_ref[...] += lax.dot_general(
        oh, emit, dimension_numbers=(((0,), (0,)), ((), ())),
        preferred_element_type=jnp.float32, precision=_HI)

    crow[...] = x[nb - 1:nb, :]
    cseg[...] = seg[nb - 1:nb, :]


def _segmax(b2, hin, N, H):
    nblk = N // _NB
    return pl.pallas_call(
        _segmax_body,
        grid=(nblk,),
        in_specs=[
            pl.BlockSpec((_NB, 1), lambda i: (i, 0)),
            pl.BlockSpec((_NB, H), lambda i: (i, 0)),
        ],
        out_specs=pl.BlockSpec((_G, H), lambda i: (0, 0)),
        out_shape=jax.ShapeDtypeStruct((_G, H), jnp.float32),
        scratch_shapes=[
            pltpu.VMEM((1, H), jnp.float32),
            pltpu.VMEM((1, 1), jnp.int32),
        ],
        compiler_params=pltpu.CompilerParams(
            dimension_semantics=("arbitrary",)),
    )(b2, hin)


def _vnup_body(vp_ref, vn_ref, w1_ref, b1_ref, g1_ref, be1_ref,
               w2_ref, b2_ref, out_ref):
    z = jnp.dot(vp_ref[...], w1_ref[...], preferred_element_type=jnp.float32)
    z = z + b1_ref[0:1, :]
    z = _ln(z, g1_ref[0:1, :], be1_ref[0:1, :])
    z = _gelu(z)
    z = jnp.dot(z, w2_ref[...], preferred_element_type=jnp.float32)
    z = z + b2_ref[0:1, :]
    out_ref[...] = vn_ref[...] + z


def _vnup(vp, vn, mlp, H):
    v = lambda a: a.reshape(1, -1).astype(jnp.float32)
    return pl.pallas_call(
        _vnup_body,
        out_shape=jax.ShapeDtypeStruct((_G, H), jnp.float32),
    )(vp, vn, mlp['W1'], v(mlp['b1']), v(mlp['g1']), v(mlp['be1']),
      mlp['W2'], v(mlp['b2']))


def _final_body(b_ref, h_ref, w1_ref, b1_ref, g_ref, be_ref, w2_ref, b2_ref,
                out_ref):
    i = pl.program_id(0)
    nblk = pl.num_programs(0)

    @pl.when(i == 0)
    def _():
        out_ref[...] = jnp.zeros_like(out_ref)

    seg = b_ref[...]
    oh = (seg == lax.broadcasted_iota(jnp.int32, (1, _G), 1)
          ).astype(jnp.float32)
    out_ref[...] += lax.dot_general(
        oh, h_ref[...], dimension_numbers=(((0,), (0,)), ((), ())),
        preferred_element_type=jnp.float32, precision=_HI)

    @pl.when(i == nblk - 1)
    def _():
        hg = out_ref[...]
        z = jnp.dot(hg, w1_ref[...], preferred_element_type=jnp.float32)
        z = z + b1_ref[0:1, :]
        z = _ln(z, g_ref[0:1, :], be_ref[0:1, :])
        z = _gelu(z)
        z = jnp.dot(z, w2_ref[...], preferred_element_type=jnp.float32)
        z = z + b2_ref[0:1, :]
        z = z / jnp.sqrt(jnp.sum(z * z, axis=-1, keepdims=True))
        out_ref[...] = z


def _final(b2, h, proj, N, H):
    nblk = N // _NB
    v = lambda a: a.reshape(1, -1).astype(jnp.float32)
    full = lambda shape: pl.BlockSpec(shape, lambda i: (0, 0))
    return pl.pallas_call(
        _final_body,
        grid=(nblk,),
        in_specs=[
            pl.BlockSpec((_NB, 1), lambda i: (i, 0)),
            pl.BlockSpec((_NB, H), lambda i: (i, 0)),
            full((H, H)),
            full((1, H)),
            full((1, H)),
            full((1, H)),
            full((H, H)),
            full((1, H)),
        ],
        out_specs=pl.BlockSpec((_G, H), lambda i: (0, 0)),
        out_shape=jax.ShapeDtypeStruct((_G, H), jnp.float32),
        compiler_params=pltpu.CompilerParams(
            dimension_semantics=("arbitrary",)),
    )(b2, h, proj['W1'], v(proj['b1']), v(proj['g']), v(proj['be']),
      proj['W2'], v(proj['b2']))


def kernel(x, edge_index, edge_attr, batch, params):
    N = x.shape[0]
    E = edge_index.shape[1]
    H = params['atom_enc'].shape[1]
    L = len(params['convs'])

    x2 = x.astype(jnp.int32)[:, None]
    b2 = batch.astype(jnp.int32)[:, None]
    src = edge_index[0].astype(jnp.int32)
    dst = edge_index[1].astype(jnp.int32)
    ea = edge_attr.astype(jnp.int32)

    gidx = ea * N + src
    ch = _edge_chunks(E)
    pad = 32 * _K * ch - E
    padidx = 5 * N + (jnp.arange(pad, dtype=jnp.int32) % 8)
    gidx2 = jnp.concatenate([gidx, padidx])
    dst2 = jnp.concatenate([dst, jnp.full((pad,), N, jnp.int32)])
    zsrc = jnp.zeros((_npad(N) // 16, H), jnp.float32)

    atom = params['atom_enc'].astype(jnp.float32)
    atom_pad = jnp.pad(atom, ((0, 128 - atom.shape[0]), (0, 0)))
    vn0 = params['vn_emb'].astype(jnp.float32)
    vn = jnp.broadcast_to(vn0[0][None, :], (_G, H)).astype(jnp.float32)

    h = None
    hin = None
    for l in range(L):
        conv = params['convs'][l]
        bond_pad = jnp.pad(conv['bond_enc'].astype(jnp.float32),
                           ((0, 3), (0, 0)))
        if l == 0:
            hin, gmsg = _pre0(x2, atom_pad, vn0, bond_pad, N, H)
        else:
            hin, gmsg = _pre(b2, h, vn, bond_pad, N, H)
        tab = jnp.concatenate(
            [gmsg.reshape(5 * N, H), jnp.zeros((8, H), jnp.float32)],
            axis=0)
        partial = _edge_agg(tab, gidx2, dst2, zsrc, N, H, E)
        h = _post(hin, partial, conv, params['norms'][l],
                  final=(l == L - 1), N=N, H=H)
        if l < L - 1:
            vp = _segmax(b2, hin, N, H)
            vn = _vnup(vp, vn, params['vn_mlps'][l], H)
    return _final(b2, h, params['proj'], N, H)
